# Initial kernel scaffold; baseline (speedup 1.0000x reference)
import jax
import jax.numpy as jnp
from jax.experimental import pallas as pl


def kernel(x, edge_index, W1, b1, W2, b2):
    raise NotImplementedError("write your pallas kernel here")


# speedup vs baseline: 25.6179x; 25.6179x over previous
import functools

import jax
import jax.numpy as jnp
from jax import lax
from jax.experimental import pallas as pl
from jax.experimental.pallas import tpu as pltpu
from jax.experimental.pallas import tpu_sc as plsc

N = 10000
NPAD = 10112
E = 320000
NC = 2
NS = 16
NW = NC * NS
K = 128
CHUNKS = 80
EP = NW * K * CHUNKS
ROWS_PER_SUB = NPAD // NS

_mesh = plsc.VectorSubcoreMesh(core_axis_name="c", subcore_axis_name="s")
_sc_params = pltpu.CompilerParams(use_tc_tiling_on_sc=False)


def _zero_shared(acc_sh, zbuf, d):
    s = lax.axis_index("s")

    @pl.loop(0, ROWS_PER_SUB)
    def _(i):
        for c in range(d // 16):
            zbuf[i, pl.ds(c * 16, 16)] = jnp.zeros((16,), jnp.float32)

    pltpu.sync_copy(zbuf, acc_sh.at[pl.ds(s * ROWS_PER_SUB, ROWS_PER_SUB)])


def _write_out(acc_sh, out_hbm):
    c = lax.axis_index("c")
    s = lax.axis_index("s")
    base = s * ROWS_PER_SUB
    pltpu.sync_copy(acc_sh.at[pl.ds(base, ROWS_PER_SUB)],
                    out_hbm.at[c].at[pl.ds(base, ROWS_PER_SUB)])


def _degree_body(dst_hbm, out_hbm, dstb, ones, zbuf, acc_sh):
    c = lax.axis_index("c")
    s = lax.axis_index("s")
    w = s * NC + c

    _zero_shared(acc_sh, zbuf, 16)

    @pl.loop(0, K)
    def _(i):
        ones[i, pl.ds(0, 16)] = jnp.ones((16,), jnp.float32)

    pltpu.sync_copy(dst_hbm.at[pl.ds(w * CHUNKS, CHUNKS)], dstb)
    plsc.subcore_barrier()

    @pl.loop(0, CHUNKS)
    def _(j):
        pltpu.sync_copy(ones, acc_sh.at[dstb.at[j]], add=True)

    plsc.subcore_barrier()
    _write_out(acc_sh, out_hbm)


def _agg_body(d, y_hbm, src_hbm, dst_hbm, out_hbm,
              srcb, dstb, rows, zbuf, acc_sh, sem):
    c = lax.axis_index("c")
    s = lax.axis_index("s")
    w = s * NC + c

    _zero_shared(acc_sh, zbuf, d)

    pltpu.sync_copy(src_hbm.at[pl.ds(w * CHUNKS, CHUNKS)], srcb)
    pltpu.sync_copy(dst_hbm.at[pl.ds(w * CHUNKS, CHUNKS)], dstb)
    plsc.subcore_barrier()

    @pl.loop(0, CHUNKS)
    def _(j):
        pltpu.async_copy(y_hbm.at[srcb.at[j]], rows, sem).wait()
        pltpu.sync_copy(rows, acc_sh.at[dstb.at[j]], add=True)

    plsc.subcore_barrier()
    _write_out(acc_sh, out_hbm)


def _sc_degree(dst2d):
    f = pl.kernel(
        _degree_body,
        out_type=jax.ShapeDtypeStruct((NC, NPAD, 16), jnp.float32),
        mesh=_mesh,
        compiler_params=_sc_params,
        scratch_types=[
            pltpu.VMEM((CHUNKS, K), jnp.int32),
            pltpu.VMEM((K, 16), jnp.float32),
            pltpu.VMEM((ROWS_PER_SUB, 16), jnp.float32),
            pltpu.VMEM_SHARED((NPAD, 16), jnp.float32),
        ],
    )
    return f(dst2d)


def _sc_aggregate(y, src2d, dst2d, d):
    f = pl.kernel(
        functools.partial(_agg_body, d),
        out_type=jax.ShapeDtypeStruct((NC, NPAD, d), jnp.float32),
        mesh=_mesh,
        compiler_params=_sc_params,
        scratch_types=[
            pltpu.VMEM((CHUNKS, K), jnp.int32),
            pltpu.VMEM((CHUNKS, K), jnp.int32),
            pltpu.VMEM((K, d), jnp.float32),
            pltpu.VMEM((ROWS_PER_SUB, d), jnp.float32),
            pltpu.VMEM_SHARED((NPAD, d), jnp.float32),
            pltpu.SemaphoreType.DMA,
        ],
    )
    return f(y, src2d, dst2d)


_BLK = 1000


def _mm1_body(x_ref, w_ref, o_ref):
    o_ref[...] = jnp.dot(x_ref[...], w_ref[...],
                         preferred_element_type=jnp.float32)


def _tc_mm1(x, W1):
    return pl.pallas_call(
        _mm1_body,
        grid=(N // _BLK,),
        in_specs=[
            pl.BlockSpec((_BLK, 128), lambda i: (i, 0)),
            pl.BlockSpec((128, 32), lambda i: (0, 0)),
        ],
        out_specs=pl.BlockSpec((_BLK, 32), lambda i: (i, 0)),
        out_shape=jax.ShapeDtypeStruct((N, 32), jnp.float32),
    )(x, W1)


def _scale1_body(degp_ref, xw_ref, y_ref, dinv_ref):
    deg = degp_ref[0, :, 0] + degp_ref[1, :, 0] + 1.0
    dinv = lax.rsqrt(deg)
    dinv_ref[...] = dinv[:, None]
    y_ref[...] = xw_ref[...] * dinv[:, None]


def _tc_scale1(degp, xw1):
    return pl.pallas_call(
        _scale1_body,
        grid=(N // _BLK,),
        in_specs=[
            pl.BlockSpec((NC, _BLK, 16), lambda i: (0, i, 0)),
            pl.BlockSpec((_BLK, 32), lambda i: (i, 0)),
        ],
        out_specs=[
            pl.BlockSpec((_BLK, 32), lambda i: (i, 0)),
            pl.BlockSpec((_BLK, 1), lambda i: (i, 0)),
        ],
        out_shape=[
            jax.ShapeDtypeStruct((N, 32), jnp.float32),
            jax.ShapeDtypeStruct((N, 1), jnp.float32),
        ],
    )(degp, xw1)


def _comb1_body(accp_ref, y_ref, dinv_ref, b1_ref, w2_ref, y2_ref):
    dinv = dinv_ref[...]
    pre = (accp_ref[0] + accp_ref[1] + y_ref[...]) * dinv + b1_ref[...]
    h = jnp.maximum(pre, 0.0)
    y2_ref[...] = jnp.dot(h, w2_ref[...],
                          preferred_element_type=jnp.float32) * dinv


def _tc_comb1(accp, y1, dinv, b1, W2):
    return pl.pallas_call(
        _comb1_body,
        grid=(N // _BLK,),
        in_specs=[
            pl.BlockSpec((NC, _BLK, 32), lambda i: (0, i, 0)),
            pl.BlockSpec((_BLK, 32), lambda i: (i, 0)),
            pl.BlockSpec((_BLK, 1), lambda i: (i, 0)),
            pl.BlockSpec((1, 32), lambda i: (0, 0)),
            pl.BlockSpec((32, 16), lambda i: (0, 0)),
        ],
        out_specs=pl.BlockSpec((_BLK, 16), lambda i: (i, 0)),
        out_shape=jax.ShapeDtypeStruct((N, 16), jnp.float32),
    )(accp, y1, dinv, b1, W2)


def _comb2_body(accp_ref, y2_ref, dinv_ref, b2_ref, o_ref):
    o_ref[...] = ((accp_ref[0] + accp_ref[1] + y2_ref[...]) * dinv_ref[...]
                  + b2_ref[...])


def _tc_comb2(accp, y2, dinv, b2):
    return pl.pallas_call(
        _comb2_body,
        grid=(N // _BLK,),
        in_specs=[
            pl.BlockSpec((NC, _BLK, 16), lambda i: (0, i, 0)),
            pl.BlockSpec((_BLK, 16), lambda i: (i, 0)),
            pl.BlockSpec((_BLK, 1), lambda i: (i, 0)),
            pl.BlockSpec((1, 16), lambda i: (0, 0)),
        ],
        out_specs=pl.BlockSpec((_BLK, 16), lambda i: (i, 0)),
        out_shape=jax.ShapeDtypeStruct((N, 16), jnp.float32),
    )(accp, y2, dinv, b2)


def kernel(x, edge_index, W1, b1, W2, b2):
    e = edge_index.astype(jnp.int32)
    npad_e = EP - E
    src_p = jnp.concatenate([e[0], jnp.zeros((npad_e,), jnp.int32)])
    dst_p = jnp.concatenate([e[1], jnp.full((npad_e,), N, jnp.int32)])
    src2d = src_p.reshape(EP // K, K)
    dst2d = dst_p.reshape(EP // K, K)
    b1r = b1.reshape(1, 32)
    b2r = b2.reshape(1, 16)

    degp = _sc_degree(dst2d)
    xw1 = _tc_mm1(x, W1)
    y1, dinv = _tc_scale1(degp, xw1)
    acc1 = _sc_aggregate(y1, src2d, dst2d, 32)
    y2 = _tc_comb1(acc1, y1, dinv, b1r, W2)
    acc2 = _sc_aggregate(y2, src2d, dst2d, 16)
    out = _tc_comb2(acc2, y2, dinv, b2r)
    return out


# speedup vs baseline: 32.4729x; 1.2676x over previous
import functools

import jax
import jax.numpy as jnp
from jax import lax
from jax.experimental import pallas as pl
from jax.experimental.pallas import tpu as pltpu
from jax.experimental.pallas import tpu_sc as plsc

N = 10000
NPAD = 10112
E = 320000
NC = 2
NS = 16
NW = NC * NS
K = 128
CHUNKS = 80
EP = NW * K * CHUNKS
ROWS_PER_SUB = NPAD // NS

_mesh = plsc.VectorSubcoreMesh(core_axis_name="c", subcore_axis_name="s")
_sc_params = pltpu.CompilerParams(use_tc_tiling_on_sc=False)


def _zero_shared(acc_sh, zbuf, d):
    s = lax.axis_index("s")

    @pl.loop(0, ROWS_PER_SUB)
    def _(i):
        for c in range(d // 16):
            zbuf[i, pl.ds(c * 16, 16)] = jnp.zeros((16,), jnp.float32)

    pltpu.sync_copy(zbuf, acc_sh.at[pl.ds(s * ROWS_PER_SUB, ROWS_PER_SUB)])


def _write_out(acc_sh, out_hbm):
    c = lax.axis_index("c")
    s = lax.axis_index("s")
    base = s * ROWS_PER_SUB
    pltpu.sync_copy(acc_sh.at[pl.ds(base, ROWS_PER_SUB)],
                    out_hbm.at[c].at[pl.ds(base, ROWS_PER_SUB)])


_NS_RING = 8


def _degree_body(dst_hbm, out_hbm, dstb, ones, zbuf, acc_sh, sems):
    c = lax.axis_index("c")
    s = lax.axis_index("s")
    w = s * NC + c

    _zero_shared(acc_sh, zbuf, 16)

    @pl.loop(0, K)
    def _(i):
        ones[i, pl.ds(0, 16)] = jnp.ones((16,), jnp.float32)

    pltpu.sync_copy(dst_hbm.at[pl.ds(w * CHUNKS, CHUNKS)], dstb)
    plsc.subcore_barrier()

    @pl.loop(0, CHUNKS // _NS_RING)
    def _(g):
        for b in range(_NS_RING):
            j = g * _NS_RING + b

            @pl.when(g > 0)
            def _():
                pltpu.make_async_copy(
                    ones, acc_sh.at[dstb.at[j]], sems.at[b]).wait()

            pltpu.async_copy(ones, acc_sh.at[dstb.at[j]], sems.at[b],
                             add=True)

    for b in range(_NS_RING):
        j = CHUNKS - _NS_RING + b
        pltpu.make_async_copy(ones, acc_sh.at[dstb.at[j]], sems.at[b]).wait()

    plsc.subcore_barrier()
    _write_out(acc_sh, out_hbm)


_NBUF = 4
_Q = 2


def _agg_body(d, y_hbm, src_hbm, dst_hbm, out_hbm,
              srcb, dstb, rows, zbuf, acc_sh, gsem, ssem):
    c = lax.axis_index("c")
    s = lax.axis_index("s")
    w = s * NC + c

    pltpu.sync_copy(src_hbm.at[pl.ds(w * CHUNKS, CHUNKS)], srcb)
    pltpu.sync_copy(dst_hbm.at[pl.ds(w * CHUNKS, CHUNKS)], dstb)

    for b in range(_Q):
        pltpu.async_copy(y_hbm.at[srcb.at[b]], rows.at[b], gsem.at[b])

    _zero_shared(acc_sh, zbuf, d)
    plsc.subcore_barrier()

    def _gather_wait(j, slot):
        pltpu.make_async_copy(y_hbm.at[srcb.at[j]], rows.at[slot],
                              gsem.at[slot]).wait()

    def _scatter_wait(j, slot):
        pltpu.make_async_copy(rows.at[slot], acc_sh.at[dstb.at[j]],
                              ssem.at[slot]).wait()

    @pl.loop(0, (CHUNKS - _Q) // _NBUF)
    def _(g):
        for b in range(_NBUF):
            jg = g * _NBUF + b + _Q
            js = g * _NBUF + b
            gslot = (b + _Q) % _NBUF
            sslot = b

            if b < _NBUF - _Q:
                @pl.when(g > 0)
                def _():
                    _scatter_wait(jg - _NBUF, gslot)
            else:
                _scatter_wait(jg - _NBUF, gslot)

            pltpu.async_copy(y_hbm.at[srcb.at[jg]], rows.at[gslot],
                             gsem.at[gslot])
            _gather_wait(js, sslot)
            pltpu.async_copy(rows.at[sslot], acc_sh.at[dstb.at[js]],
                             ssem.at[sslot], add=True)

    for i in range(_Q):
        jg = CHUNKS - _Q + i
        gslot = jg % _NBUF
        _scatter_wait(jg - _NBUF, gslot)
        pltpu.async_copy(y_hbm.at[srcb.at[jg]], rows.at[gslot],
                         gsem.at[gslot])
    for i in range(_NBUF):
        js = CHUNKS - _NBUF + i
        sslot = js % _NBUF
        _gather_wait(js, sslot)
        pltpu.async_copy(rows.at[sslot], acc_sh.at[dstb.at[js]],
                         ssem.at[sslot], add=True)
    for i in range(_NBUF):
        js = CHUNKS - _NBUF + i
        _scatter_wait(js, js % _NBUF)

    plsc.subcore_barrier()
    _write_out(acc_sh, out_hbm)


def _sc_degree(dst2d):
    f = pl.kernel(
        _degree_body,
        out_type=jax.ShapeDtypeStruct((NC, NPAD, 16), jnp.float32),
        mesh=_mesh,
        compiler_params=_sc_params,
        scratch_types=[
            pltpu.VMEM((CHUNKS, K), jnp.int32),
            pltpu.VMEM((K, 16), jnp.float32),
            pltpu.VMEM((ROWS_PER_SUB, 16), jnp.float32),
            pltpu.VMEM_SHARED((NPAD, 16), jnp.float32),
            pltpu.SemaphoreType.DMA((_NS_RING,)),
        ],
    )
    return f(dst2d)


def _sc_aggregate(y, src2d, dst2d, d):
    f = pl.kernel(
        functools.partial(_agg_body, d),
        out_type=jax.ShapeDtypeStruct((NC, NPAD, d), jnp.float32),
        mesh=_mesh,
        compiler_params=_sc_params,
        scratch_types=[
            pltpu.VMEM((CHUNKS, K), jnp.int32),
            pltpu.VMEM((CHUNKS, K), jnp.int32),
            pltpu.VMEM((_NBUF, K, d), jnp.float32),
            pltpu.VMEM((ROWS_PER_SUB, d), jnp.float32),
            pltpu.VMEM_SHARED((NPAD, d), jnp.float32),
            pltpu.SemaphoreType.DMA((_NBUF,)),
            pltpu.SemaphoreType.DMA((_NBUF,)),
        ],
    )
    return f(y, src2d, dst2d)


_BLK = 1000


def _mm1_body(x_ref, w_ref, o_ref):
    o_ref[...] = jnp.dot(x_ref[...], w_ref[...],
                         preferred_element_type=jnp.float32)


def _tc_mm1(x, W1):
    return pl.pallas_call(
        _mm1_body,
        grid=(N // _BLK,),
        in_specs=[
            pl.BlockSpec((_BLK, 128), lambda i: (i, 0)),
            pl.BlockSpec((128, 32), lambda i: (0, 0)),
        ],
        out_specs=pl.BlockSpec((_BLK, 32), lambda i: (i, 0)),
        out_shape=jax.ShapeDtypeStruct((N, 32), jnp.float32),
    )(x, W1)


def _scale1_body(degp_ref, xw_ref, y_ref, dinv_ref):
    deg = degp_ref[0, :, 0] + degp_ref[1, :, 0] + 1.0
    dinv = lax.rsqrt(deg)
    dinv_ref[...] = dinv[:, None]
    y_ref[...] = xw_ref[...] * dinv[:, None]


def _tc_scale1(degp, xw1):
    return pl.pallas_call(
        _scale1_body,
        grid=(N // _BLK,),
        in_specs=[
            pl.BlockSpec((NC, _BLK, 16), lambda i: (0, i, 0)),
            pl.BlockSpec((_BLK, 32), lambda i: (i, 0)),
        ],
        out_specs=[
            pl.BlockSpec((_BLK, 32), lambda i: (i, 0)),
            pl.BlockSpec((_BLK, 1), lambda i: (i, 0)),
        ],
        out_shape=[
            jax.ShapeDtypeStruct((N, 32), jnp.float32),
            jax.ShapeDtypeStruct((N, 1), jnp.float32),
        ],
    )(degp, xw1)


def _comb1_body(accp_ref, y_ref, dinv_ref, b1_ref, w2_ref, y2_ref):
    dinv = dinv_ref[...]
    pre = (accp_ref[0] + accp_ref[1] + y_ref[...]) * dinv + b1_ref[...]
    h = jnp.maximum(pre, 0.0)
    y2_ref[...] = jnp.dot(h, w2_ref[...],
                          preferred_element_type=jnp.float32) * dinv


def _tc_comb1(accp, y1, dinv, b1, W2):
    return pl.pallas_call(
        _comb1_body,
        grid=(N // _BLK,),
        in_specs=[
            pl.BlockSpec((NC, _BLK, 32), lambda i: (0, i, 0)),
            pl.BlockSpec((_BLK, 32), lambda i: (i, 0)),
            pl.BlockSpec((_BLK, 1), lambda i: (i, 0)),
            pl.BlockSpec((1, 32), lambda i: (0, 0)),
            pl.BlockSpec((32, 16), lambda i: (0, 0)),
        ],
        out_specs=pl.BlockSpec((_BLK, 16), lambda i: (i, 0)),
        out_shape=jax.ShapeDtypeStruct((N, 16), jnp.float32),
    )(accp, y1, dinv, b1, W2)


def _comb2_body(accp_ref, y2_ref, dinv_ref, b2_ref, o_ref):
    o_ref[...] = ((accp_ref[0] + accp_ref[1] + y2_ref[...]) * dinv_ref[...]
                  + b2_ref[...])


def _tc_comb2(accp, y2, dinv, b2):
    return pl.pallas_call(
        _comb2_body,
        grid=(N // _BLK,),
        in_specs=[
            pl.BlockSpec((NC, _BLK, 16), lambda i: (0, i, 0)),
            pl.BlockSpec((_BLK, 16), lambda i: (i, 0)),
            pl.BlockSpec((_BLK, 1), lambda i: (i, 0)),
            pl.BlockSpec((1, 16), lambda i: (0, 0)),
        ],
        out_specs=pl.BlockSpec((_BLK, 16), lambda i: (i, 0)),
        out_shape=jax.ShapeDtypeStruct((N, 16), jnp.float32),
    )(accp, y2, dinv, b2)


def kernel(x, edge_index, W1, b1, W2, b2):
    e = edge_index.astype(jnp.int32)
    npad_e = EP - E
    src_p = jnp.concatenate([e[0], jnp.zeros((npad_e,), jnp.int32)])
    dst_p = jnp.concatenate([e[1], jnp.full((npad_e,), N, jnp.int32)])
    src2d = src_p.reshape(EP // K, K)
    dst2d = dst_p.reshape(EP // K, K)
    b1r = b1.reshape(1, 32)
    b2r = b2.reshape(1, 16)

    degp = _sc_degree(dst2d)
    xw1 = _tc_mm1(x, W1)
    y1, dinv = _tc_scale1(degp, xw1)
    acc1 = _sc_aggregate(y1, src2d, dst2d, 32)
    y2 = _tc_comb1(acc1, y1, dinv, b1r, W2)
    acc2 = _sc_aggregate(y2, src2d, dst2d, 16)
    out = _tc_comb2(acc2, y2, dinv, b2r)
    return out
